# baseline (speedup 1.0000x reference)
import functools

import jax
import jax.numpy as jnp
from jax import lax
from jax.experimental import pallas as pl
from jax.experimental.pallas import tpu as pltpu
from jax.experimental.pallas import tpu_sc as plsc

T = 2048
D = 768
E = 8
H = 4 * D
TILE_M = 128
NT = T // TILE_M
STEPS = NT + E - 1
NBLK = 3072
NB = H // NBLK
NW = 32
BPW = T // NW


_RC = 256


def _router_body(x_ref, wr_ref, pos_ref, cnt_ref, m_scr):
    logits = jnp.dot(x_ref[...], wr_ref[...], preferred_element_type=jnp.float32)
    m = jnp.max(logits, axis=1, keepdims=True)
    col = lax.broadcasted_iota(jnp.int32, logits.shape, 1)
    idx = jnp.min(jnp.where(logits >= m, col, E), axis=1, keepdims=True)
    oneh = (col == idx).astype(jnp.float32)
    m_scr[...] = oneh
    cnt = jnp.sum(oneh, axis=0, keepdims=True)
    tri = (lax.broadcasted_iota(jnp.int32, (E, E), 0)
           < lax.broadcasted_iota(jnp.int32, (E, E), 1)).astype(jnp.float32)
    offs = jnp.dot(cnt, tri, preferred_element_type=jnp.float32)
    cnt_ref[...] = jnp.broadcast_to(cnt, (8, E)).astype(jnp.int32)

    def chunk(i, _):
        base = i * _RC
        rowi = lax.broadcasted_iota(jnp.int32, (_RC, T), 0) + base
        colt = lax.broadcasted_iota(jnp.int32, (_RC, T), 1)
        ltri = (colt < rowi).astype(jnp.float32)
        rank = jnp.dot(ltri, m_scr[...], preferred_element_type=jnp.float32)
        mch = m_scr[pl.ds(base, _RC), :]
        pos = jnp.sum((rank + offs) * mch, axis=1, keepdims=True)
        pos_ref[pl.ds(base, _RC), :] = jnp.broadcast_to(
            pos.astype(jnp.int32), (_RC, E))
        return 0

    lax.fori_loop(0, T // _RC, chunk, 0)


def _router(x, Wr):
    pos, cnt = pl.pallas_call(
        _router_body,
        out_shape=(jax.ShapeDtypeStruct((T, E), jnp.int32),
                   jax.ShapeDtypeStruct((8, E), jnp.int32)),
        scratch_shapes=[pltpu.VMEM((T, E), jnp.float32)],
    )(x, Wr)
    return pos[:, 0], cnt[0]


def _mlp_body(tile_ref, eof_ref, lo_ref, hi_ref, valid_ref, first_ref,
              nmap_ref, xs_ref, wg_ref, wu_ref, cp_ref, o_ref):
    s = pl.program_id(0)
    n = pl.program_id(1)

    @pl.when(valid_ref[s] == 1)
    def _():
        gid = tile_ref[s] * TILE_M + lax.broadcasted_iota(
            jnp.int32, (TILE_M, 1), 0)
        msk = ((gid >= lo_ref[s]) & (gid < hi_ref[s])).astype(jnp.bfloat16)
        xt = xs_ref[...].astype(jnp.bfloat16) * msk
        wg = wg_ref[0].astype(jnp.bfloat16)
        wu = wu_ref[0].astype(jnp.bfloat16)
        cp = cp_ref[0].astype(jnp.bfloat16)
        g = jnp.dot(xt, wg, preferred_element_type=jnp.float32)
        u = jnp.dot(xt, wu, preferred_element_type=jnp.float32)
        a = (g * jax.nn.sigmoid(g) * u).astype(jnp.bfloat16)
        part = jnp.dot(a, cp, preferred_element_type=jnp.float32)
        init = (n == 0) & (first_ref[s] == 1)

        @pl.when(init)
        def _():
            o_ref[...] = part

        @pl.when(jnp.logical_not(init))
        def _():
            o_ref[...] += part


def _grouped_mlp(xs, w_v, c_proj, sched):
    grid_spec = pltpu.PrefetchScalarGridSpec(
        num_scalar_prefetch=7,
        grid=(STEPS, NB),
        in_specs=[
            pl.BlockSpec((TILE_M, D),
                         lambda s, n, tile, eof, lo, hi, valid, first, nmap:
                         (tile[s], 0)),
            pl.BlockSpec((1, D, NBLK),
                         lambda s, n, tile, eof, lo, hi, valid, first, nmap:
                         (eof[s], 0, nmap[s, n])),
            pl.BlockSpec((1, D, NBLK),
                         lambda s, n, tile, eof, lo, hi, valid, first, nmap:
                         (eof[s], 0, NB + nmap[s, n])),
            pl.BlockSpec((1, NBLK, D),
                         lambda s, n, tile, eof, lo, hi, valid, first, nmap:
                         (eof[s], nmap[s, n], 0)),
        ],
        out_specs=pl.BlockSpec((TILE_M, D),
                               lambda s, n, tile, eof, lo, hi, valid, first,
                               nmap: (tile[s], 0)),
    )
    return pl.pallas_call(
        _mlp_body,
        grid_spec=grid_spec,
        out_shape=jax.ShapeDtypeStruct((T, D), jnp.float32),
        compiler_params=pltpu.CompilerParams(
            dimension_semantics=("arbitrary", "arbitrary")),
    )(*sched, xs, w_v, w_v, c_proj)


def _sc_gather(x, gidx):
    mesh = plsc.VectorSubcoreMesh(core_axis_name="c", subcore_axis_name="s")

    @functools.partial(
        pl.kernel,
        mesh=mesh,
        out_type=jax.ShapeDtypeStruct((T, D), jnp.float32),
        scratch_types=[
            pltpu.VMEM((BPW,), jnp.int32),
            pltpu.VMEM((BPW, D), jnp.float32),
            pltpu.SemaphoreType.DMA,
        ],
    )
    def k(x_hbm, idx_hbm, out_hbm, idx_v, rows_v, sem):
        wid = lax.axis_index("s") * 2 + lax.axis_index("c")
        base = wid * BPW
        pltpu.sync_copy(idx_hbm.at[pl.ds(base, BPW)], idx_v)
        pltpu.async_copy(x_hbm.at[idx_v], rows_v, sem).wait()
        pltpu.sync_copy(rows_v, out_hbm.at[pl.ds(base, BPW)])

    return k(x, gidx)


def _sc_scatter(y, sidx):
    mesh = plsc.VectorSubcoreMesh(core_axis_name="c", subcore_axis_name="s")

    @functools.partial(
        pl.kernel,
        mesh=mesh,
        out_type=jax.ShapeDtypeStruct((T, D), jnp.float32),
        scratch_types=[
            pltpu.VMEM((BPW,), jnp.int32),
            pltpu.VMEM((BPW, D), jnp.float32),
            pltpu.SemaphoreType.DMA,
        ],
    )
    def k(y_hbm, idx_hbm, out_hbm, idx_v, rows_v, sem):
        wid = lax.axis_index("s") * 2 + lax.axis_index("c")
        base = wid * BPW
        pltpu.sync_copy(idx_hbm.at[pl.ds(base, BPW)], idx_v)
        pltpu.sync_copy(y_hbm.at[pl.ds(base, BPW)], rows_v)
        pltpu.async_copy(rows_v, out_hbm.at[idx_v], sem).wait()

    return k(y, sidx)


def _schedule(cnt):
    offs = (jnp.cumsum(cnt) - cnt).astype(jnp.int32)
    cum_end = offs + cnt

    cuts = jnp.sort(jnp.concatenate(
        [jnp.arange(NT, dtype=jnp.int32) * TILE_M, offs[1:]]))
    lo = cuts
    hi = jnp.concatenate([cuts[1:], jnp.full((1,), T, jnp.int32)])
    validv = hi > lo
    eraw = jnp.clip(jnp.searchsorted(cum_end, lo, side="right"),
                    0, E - 1).astype(jnp.int32)
    traw = jnp.clip(lo // TILE_M, 0, NT - 1)

    v = jnp.arange(STEPS, dtype=jnp.int32)
    prev = jnp.clip(lax.cummax(jnp.where(validv, v, -1)), 0, STEPS - 1)
    eof = jnp.where(validv, eraw, eraw[prev])
    tile = jnp.where(validv, traw, traw[prev])

    same_before = ((eraw[None, :] == eraw[:, None]) & validv[None, :]
                   & (v[None, :] < v[:, None]))
    dirn = (jnp.sum(same_before, axis=1) % 2).astype(jnp.int32)
    final_neff = jnp.where(dirn == 0, NB - 1, 0)
    narr = jnp.broadcast_to(jnp.arange(NB, dtype=jnp.int32)[None, :],
                            (STEPS, NB))
    nmap = jnp.where(validv[:, None],
                     jnp.where(dirn[:, None] == 1, NB - 1 - narr, narr),
                     final_neff[prev][:, None])
    first = (validv & (lo % TILE_M == 0)).astype(jnp.int32)
    valid = validv.astype(jnp.int32)
    return (tile, eof, lo, hi, valid, first, nmap)


def kernel(x, Wr, w_v, c_proj):
    pos, cnt = _router(x, Wr)
    sched = _schedule(cnt)
    xs = _sc_scatter(x, pos)
    y = _grouped_mlp(xs, w_v, c_proj, sched)
    return _sc_gather(y, pos)


# speedup vs baseline: 1.0011x; 1.0011x over previous
import functools

import jax
import jax.numpy as jnp
from jax import lax
from jax.experimental import pallas as pl
from jax.experimental.pallas import tpu as pltpu
from jax.experimental.pallas import tpu_sc as plsc

T = 2048
D = 768
E = 8
H = 4 * D
TILE_M = 128
NT = T // TILE_M
STEPS = NT + E - 1
NBLK = 3072
NB = H // NBLK
NW = 32
BPW = T // NW


_RC = 256


def _router_body(x_ref, wr_ref, pos_ref, cnt_ref, m_scr):
    logits = jnp.dot(x_ref[...], wr_ref[...], preferred_element_type=jnp.float32)
    m = jnp.max(logits, axis=1, keepdims=True)
    col = lax.broadcasted_iota(jnp.int32, logits.shape, 1)
    idx = jnp.min(jnp.where(logits >= m, col, E), axis=1, keepdims=True)
    oneh = (col == idx).astype(jnp.float32)
    m_scr[...] = oneh
    cnt = jnp.sum(oneh, axis=0, keepdims=True)
    tri = (lax.broadcasted_iota(jnp.int32, (E, E), 0)
           < lax.broadcasted_iota(jnp.int32, (E, E), 1)).astype(jnp.float32)
    pre = jnp.dot(oneh, tri, preferred_element_type=jnp.float32)
    offs = jnp.sum(pre, axis=0, keepdims=True)
    cnt_ref[...] = jnp.broadcast_to(cnt, (8, E)).astype(jnp.int32)

    def chunk(i, _):
        base = i * _RC
        rowi = lax.broadcasted_iota(jnp.int32, (_RC, T), 0) + base
        colt = lax.broadcasted_iota(jnp.int32, (_RC, T), 1)
        ltri = (colt < rowi).astype(jnp.float32)
        rank = jnp.dot(ltri, m_scr[...], preferred_element_type=jnp.float32)
        mch = m_scr[pl.ds(base, _RC), :]
        pos = jnp.sum((rank + offs) * mch, axis=1, keepdims=True)
        pos_ref[pl.ds(base, _RC), :] = jnp.broadcast_to(
            pos.astype(jnp.int32), (_RC, E))
        return 0

    lax.fori_loop(0, T // _RC, chunk, 0)


def _router(x, Wr):
    pos, cnt = pl.pallas_call(
        _router_body,
        out_shape=(jax.ShapeDtypeStruct((T, E), jnp.int32),
                   jax.ShapeDtypeStruct((8, E), jnp.int32)),
        scratch_shapes=[pltpu.VMEM((T, E), jnp.float32)],
    )(x, Wr)
    return pos[:, 0], cnt[0]


def _mlp_body(tile_ref, eof_ref, lo_ref, hi_ref, valid_ref, first_ref,
              nmap_ref, xs_ref, wg_ref, wu_ref, cp_ref, o_ref):
    s = pl.program_id(0)
    n = pl.program_id(1)

    @pl.when(valid_ref[s] == 1)
    def _():
        gid = tile_ref[s] * TILE_M + lax.broadcasted_iota(
            jnp.int32, (TILE_M, 1), 0)
        msk = ((gid >= lo_ref[s]) & (gid < hi_ref[s])).astype(jnp.bfloat16)
        xt = xs_ref[...].astype(jnp.bfloat16) * msk
        wg = wg_ref[0].astype(jnp.bfloat16)
        wu = wu_ref[0].astype(jnp.bfloat16)
        cp = cp_ref[0].astype(jnp.bfloat16)
        g = jnp.dot(xt, wg, preferred_element_type=jnp.float32)
        u = jnp.dot(xt, wu, preferred_element_type=jnp.float32)
        a = (g * jax.nn.sigmoid(g) * u).astype(jnp.bfloat16)
        part = jnp.dot(a, cp, preferred_element_type=jnp.float32)
        init = (n == 0) & (first_ref[s] == 1)

        @pl.when(init)
        def _():
            o_ref[...] = part

        @pl.when(jnp.logical_not(init))
        def _():
            o_ref[...] += part


def _grouped_mlp(xs, w_v, c_proj, sched):
    grid_spec = pltpu.PrefetchScalarGridSpec(
        num_scalar_prefetch=7,
        grid=(STEPS, NB),
        in_specs=[
            pl.BlockSpec((TILE_M, D),
                         lambda s, n, tile, eof, lo, hi, valid, first, nmap:
                         (tile[s], 0)),
            pl.BlockSpec((1, D, NBLK),
                         lambda s, n, tile, eof, lo, hi, valid, first, nmap:
                         (eof[s], 0, nmap[s, n])),
            pl.BlockSpec((1, D, NBLK),
                         lambda s, n, tile, eof, lo, hi, valid, first, nmap:
                         (eof[s], 0, NB + nmap[s, n])),
            pl.BlockSpec((1, NBLK, D),
                         lambda s, n, tile, eof, lo, hi, valid, first, nmap:
                         (eof[s], nmap[s, n], 0)),
        ],
        out_specs=pl.BlockSpec((TILE_M, D),
                               lambda s, n, tile, eof, lo, hi, valid, first,
                               nmap: (tile[s], 0)),
    )
    return pl.pallas_call(
        _mlp_body,
        grid_spec=grid_spec,
        out_shape=jax.ShapeDtypeStruct((T, D), jnp.float32),
        compiler_params=pltpu.CompilerParams(
            dimension_semantics=("arbitrary", "arbitrary")),
    )(*sched, xs, w_v, w_v, c_proj)


def _sc_gather(x, gidx):
    mesh = plsc.VectorSubcoreMesh(core_axis_name="c", subcore_axis_name="s")

    @functools.partial(
        pl.kernel,
        mesh=mesh,
        out_type=jax.ShapeDtypeStruct((T, D), jnp.float32),
        scratch_types=[
            pltpu.VMEM((BPW,), jnp.int32),
            pltpu.VMEM((BPW, D), jnp.float32),
            pltpu.SemaphoreType.DMA,
        ],
    )
    def k(x_hbm, idx_hbm, out_hbm, idx_v, rows_v, sem):
        wid = lax.axis_index("s") * 2 + lax.axis_index("c")
        base = wid * BPW
        pltpu.sync_copy(idx_hbm.at[pl.ds(base, BPW)], idx_v)
        pltpu.async_copy(x_hbm.at[idx_v], rows_v, sem).wait()
        pltpu.sync_copy(rows_v, out_hbm.at[pl.ds(base, BPW)])

    return k(x, gidx)


def _sc_scatter(y, sidx):
    mesh = plsc.VectorSubcoreMesh(core_axis_name="c", subcore_axis_name="s")

    @functools.partial(
        pl.kernel,
        mesh=mesh,
        out_type=jax.ShapeDtypeStruct((T, D), jnp.float32),
        scratch_types=[
            pltpu.VMEM((BPW,), jnp.int32),
            pltpu.VMEM((BPW, D), jnp.float32),
            pltpu.SemaphoreType.DMA,
        ],
    )
    def k(y_hbm, idx_hbm, out_hbm, idx_v, rows_v, sem):
        wid = lax.axis_index("s") * 2 + lax.axis_index("c")
        base = wid * BPW
        pltpu.sync_copy(idx_hbm.at[pl.ds(base, BPW)], idx_v)
        pltpu.sync_copy(y_hbm.at[pl.ds(base, BPW)], rows_v)
        pltpu.async_copy(rows_v, out_hbm.at[idx_v], sem).wait()

    return k(y, sidx)


def _schedule(cnt):
    offs = (jnp.cumsum(cnt) - cnt).astype(jnp.int32)
    cum_end = offs + cnt

    cuts = jnp.sort(jnp.concatenate(
        [jnp.arange(NT, dtype=jnp.int32) * TILE_M, offs[1:]]))
    lo = cuts
    hi = jnp.concatenate([cuts[1:], jnp.full((1,), T, jnp.int32)])
    validv = hi > lo
    eraw = jnp.clip(jnp.searchsorted(cum_end, lo, side="right"),
                    0, E - 1).astype(jnp.int32)
    traw = jnp.clip(lo // TILE_M, 0, NT - 1)

    v = jnp.arange(STEPS, dtype=jnp.int32)
    prev = jnp.clip(lax.cummax(jnp.where(validv, v, -1)), 0, STEPS - 1)
    eof = jnp.where(validv, eraw, eraw[prev])
    tile = jnp.where(validv, traw, traw[prev])

    same_before = ((eraw[None, :] == eraw[:, None]) & validv[None, :]
                   & (v[None, :] < v[:, None]))
    dirn = (jnp.sum(same_before, axis=1) % 2).astype(jnp.int32)
    final_neff = jnp.where(dirn == 0, NB - 1, 0)
    narr = jnp.broadcast_to(jnp.arange(NB, dtype=jnp.int32)[None, :],
                            (STEPS, NB))
    nmap = jnp.where(validv[:, None],
                     jnp.where(dirn[:, None] == 1, NB - 1 - narr, narr),
                     final_neff[prev][:, None])
    first = (validv & (lo % TILE_M == 0)).astype(jnp.int32)
    valid = validv.astype(jnp.int32)
    return (tile, eof, lo, hi, valid, first, nmap)


def kernel(x, Wr, w_v, c_proj):
    pos, cnt = _router(x, Wr)
    sched = _schedule(cnt)
    xs = _sc_scatter(x, pos)
    y = _grouped_mlp(xs, w_v, c_proj, sched)
    return _sc_gather(y, pos)
